# Initial kernel scaffold; baseline (speedup 1.0000x reference)
import functools

import jax
import jax.numpy as jnp
from jax import lax
from jax.experimental import pallas as pl
from jax.experimental.pallas import tpu as pltpu
from jax.experimental.pallas import tpu_sc as plsc

N_ROWS = 100000
D = 64
B = 4096
L = 50

NC, NS = 2, 16
NW = NC * NS
BPW = B // NW

_NEG = -1e30
_GROUPS = (0, 16, 32, 34)


def _scal_body(ei_ref, eu_ref, w2, w3, w5, w6, w8, w9, iscal_ref, uscal_ref):
    m23 = w2[...] @ w3[...]
    m89 = w8[...] @ w9[...]
    m56 = w5[...] @ w6[...]
    mi = jnp.concatenate([m23, w3[...], m89, w9[...]], axis=1)
    iscal_ref[...] = ei_ref[...] @ mi
    uscal_ref[...] = eu_ref[...] @ m56


def _scal_tables(embed_item, embed_user, W2, W3, W5, W6, W8, W9):
    nb = 5000
    grid = N_ROWS // nb
    full = lambda shape: pl.BlockSpec(shape, lambda i: (0, 0))
    return pl.pallas_call(
        _scal_body,
        grid=(grid,),
        in_specs=[
            pl.BlockSpec((nb, D), lambda i: (i, 0)),
            pl.BlockSpec((nb, D), lambda i: (i, 0)),
            full((D, D)), full((D, 1)), full((D, D)),
            full((D, 1)), full((D, D)), full((D, 1)),
        ],
        out_specs=[
            pl.BlockSpec((nb, 4), lambda i: (i, 0)),
            pl.BlockSpec((nb, 1), lambda i: (i, 0)),
        ],
        out_shape=[
            jax.ShapeDtypeStruct((N_ROWS, 4), jnp.float32),
            jax.ShapeDtypeStruct((N_ROWS, 1), jnp.float32),
        ],
    )(embed_item, embed_user, W2, W3, W5, W6, W8, W9)


def _attention(scal_ref, col_a, col_b, r_ref, b):
    lane = lax.iota(jnp.int32, 16)
    svecs = []
    for off in _GROUPS:
        l_idx = lane + off
        a = plsc.load_gather(scal_ref, [l_idx, jnp.zeros((16,), jnp.int32) + col_a])
        if r_ref is not None:
            bb = plsc.load_gather(scal_ref, [l_idx, jnp.zeros((16,), jnp.int32) + col_b])
            r = r_ref[b, pl.ds(off, 16)]
            s = a + r * bb
        else:
            s = a
        if off == 34:
            s = jnp.where(lane >= 14, s, jnp.float32(_NEG))
        svecs.append(s)
    mv = jnp.maximum(jnp.maximum(svecs[0], svecs[1]),
                     jnp.maximum(svecs[2], svecs[3]))
    m = jnp.max(mv)
    evecs = [jnp.exp(s - m) for s in svecs]
    total = jnp.sum(evecs[0] + evecs[1] + evecs[2] + evecs[3])
    inv = 1.0 / total
    return evecs, inv


def _weighted_sum(e_ref, w_ref, inv):
    acc = [jnp.zeros((16,), jnp.float32) for _ in range(4)]
    for l in range(L):
        wl = w_ref[l]
        for d in range(4):
            acc[d] = acc[d] + wl * e_ref[l, pl.ds(16 * d, 16)]
    return [a * inv for a in acc]


def _sc_body(uid_hbm, iid_hbm, hui_hbm, hur_hbm, soc_hbm, hii_hbm, hir_hbm,
             eu_hbm, ei_hbm, iscal_hbm, uscal_hbm,
             h_hbm,
             uid_v, iid_v, ui_v, ur_v, fr_v, ii_v, ir_v, ue_v, ie_v, h_v,
             e1, s1, e2, s2, e3, s3, w_v, sem):
    wid = lax.axis_index("s") * NC + lax.axis_index("c")
    base = wid * BPW

    pltpu.sync_copy(uid_hbm.at[pl.ds(base, BPW)], uid_v)
    pltpu.sync_copy(iid_hbm.at[pl.ds(base, BPW)], iid_v)
    hs = [
        pltpu.async_copy(hui_hbm.at[uid_v], ui_v, sem),
        pltpu.async_copy(hur_hbm.at[uid_v], ur_v, sem),
        pltpu.async_copy(soc_hbm.at[uid_v], fr_v, sem),
        pltpu.async_copy(hii_hbm.at[iid_v], ii_v, sem),
        pltpu.async_copy(hir_hbm.at[iid_v], ir_v, sem),
        pltpu.async_copy(eu_hbm.at[uid_v], ue_v, sem),
        pltpu.async_copy(ei_hbm.at[iid_v], ie_v, sem),
    ]
    for h in hs:
        h.wait()

    def body(b, carry):
        gs = [
            pltpu.async_copy(ei_hbm.at[ui_v.at[b]], e1, sem),
            pltpu.async_copy(iscal_hbm.at[ui_v.at[b]], s1, sem),
            pltpu.async_copy(eu_hbm.at[fr_v.at[b]], e2, sem),
            pltpu.async_copy(uscal_hbm.at[fr_v.at[b]], s2, sem),
            pltpu.async_copy(ei_hbm.at[ii_v.at[b]], e3, sem),
            pltpu.async_copy(iscal_hbm.at[ii_v.at[b]], s3, sem),
        ]
        for h in gs:
            h.wait()

        outs = []
        for e_ref, sc_ref, ca, cb, r_ref in (
                (e1, s1, 0, 1, ur_v),
                (e2, s2, 0, 0, None),
                (e3, s3, 2, 3, ir_v)):
            evecs, inv = _attention(sc_ref, ca, cb, r_ref, b)
            w_v[pl.ds(34, 16)] = evecs[3]
            w_v[pl.ds(0, 16)] = evecs[0]
            w_v[pl.ds(16, 16)] = evecs[1]
            w_v[pl.ds(32, 16)] = evecs[2]
            outs.append(_weighted_sum(e_ref, w_v, inv))

        o1, o2, o3 = outs
        for d in range(4):
            sl = pl.ds(16 * d, 16)
            h_v[b, sl] = ue_v[b, sl] + o1[d] + o2[d]
            h_v[b, pl.ds(D + 16 * d, 16)] = ie_v[b, sl] + o3[d]
        return carry

    lax.fori_loop(0, BPW, body, 0)
    pltpu.sync_copy(h_v, h_hbm.at[pl.ds(base, BPW)])


def _sc_attend(user_ids, item_ids, hui, hur, soc, hii, hir,
               embed_user, embed_item, item_scal, user_scal):
    mesh = plsc.VectorSubcoreMesh(core_axis_name="c", subcore_axis_name="s")
    f32, i32 = jnp.float32, jnp.int32
    kern = pl.kernel(
        _sc_body,
        out_type=jax.ShapeDtypeStruct((B, 2 * D), f32),
        mesh=mesh,
        scratch_types=[
            pltpu.VMEM((BPW,), i32), pltpu.VMEM((BPW,), i32),
            pltpu.VMEM((BPW, L), i32), pltpu.VMEM((BPW, L), f32),
            pltpu.VMEM((BPW, L), i32), pltpu.VMEM((BPW, L), i32),
            pltpu.VMEM((BPW, L), f32),
            pltpu.VMEM((BPW, D), f32), pltpu.VMEM((BPW, D), f32),
            pltpu.VMEM((BPW, 2 * D), f32),
            pltpu.VMEM((L, D), f32), pltpu.VMEM((L, 4), f32),
            pltpu.VMEM((L, D), f32), pltpu.VMEM((L, 1), f32),
            pltpu.VMEM((L, D), f32), pltpu.VMEM((L, 4), f32),
            pltpu.VMEM((D,), f32),
            pltpu.SemaphoreType.DMA,
        ],
    )
    return kern(user_ids, item_ids, hui, hur, soc, hii, hir,
                embed_user, embed_item, item_scal, user_scal)


def _mlp_body(h_ref, w1, b1, w2, b2, o_ref):
    hh = jnp.maximum(h_ref[...] @ w1[...] + b1[...], 0.0)
    o_ref[...] = hh @ w2[...] + b2[...]


def _mlp(h, fc1_w, fc1_b, fc2_w, fc2_b):
    out = pl.pallas_call(
        _mlp_body,
        out_shape=jax.ShapeDtypeStruct((B, 1), jnp.float32),
    )(h, fc1_w, fc1_b.reshape(1, D), fc2_w, fc2_b.reshape(1, 1))
    return out.reshape(B)


def kernel(user_ids, item_ids, embed_user, embed_item, hist_u_items,
           hist_u_ratings, social_nbrs, hist_i_items, hist_i_ratings,
           W1, W2, W3, W4, W5, W6, W7, W8, W9, fc1_w, fc1_b, fc2_w, fc2_b):
    i32 = jnp.int32
    item_scal, user_scal = _scal_tables(
        embed_item, embed_user, W2, W3, W5, W6, W8, W9)
    h = _sc_attend(user_ids.astype(i32), item_ids.astype(i32),
                   hist_u_items.astype(i32), hist_u_ratings,
                   social_nbrs.astype(i32), hist_i_items.astype(i32),
                   hist_i_ratings, embed_user, embed_item,
                   item_scal, user_scal)
    return _mlp(h, fc1_w, fc1_b, fc2_w, fc2_b)


# speedup vs baseline: 3.1725x; 3.1725x over previous
import functools

import jax
import jax.numpy as jnp
from jax import lax
from jax.experimental import pallas as pl
from jax.experimental.pallas import tpu as pltpu
from jax.experimental.pallas import tpu_sc as plsc

N_ROWS = 100000
D = 64
B = 4096
L = 50

NC, NS = 2, 16
NW = NC * NS
BPW = B // NW

_NEG = -1e30
_GROUPS = (0, 16, 32, 34)


def _scal_body(ei_ref, eu_ref, w2, w3, w5, w6, w8, w9, scal_ref):
    m23 = w2[...] @ w3[...]
    m89 = w8[...] @ w9[...]
    m56 = w5[...] @ w6[...]
    mi = jnp.concatenate([m23, w3[...], m89, w9[...]], axis=1)
    nb = ei_ref.shape[0]
    scal_ref[...] = jnp.concatenate(
        [ei_ref[...] @ mi, eu_ref[...] @ m56,
         jnp.zeros((nb, 3), jnp.float32)], axis=1)


def _scal_tables(embed_item, embed_user, W2, W3, W5, W6, W8, W9):
    nb = 5000
    grid = N_ROWS // nb
    full = lambda shape: pl.BlockSpec(shape, lambda i: (0, 0))
    return pl.pallas_call(
        _scal_body,
        grid=(grid,),
        in_specs=[
            pl.BlockSpec((nb, D), lambda i: (i, 0)),
            pl.BlockSpec((nb, D), lambda i: (i, 0)),
            full((D, D)), full((D, 1)), full((D, D)),
            full((D, 1)), full((D, D)), full((D, 1)),
        ],
        out_specs=pl.BlockSpec((nb, 8), lambda i: (i, 0)),
        out_shape=jax.ShapeDtypeStruct((N_ROWS, 8), jnp.float32),
    )(embed_item, embed_user, W2, W3, W5, W6, W8, W9)


def _attention(scal_ref, col_a, col_b, r_ref):
    lane = lax.iota(jnp.int32, 16)
    svecs = []
    for off in _GROUPS:
        l_idx = lane + off
        a = plsc.load_gather(scal_ref, [l_idx, jnp.zeros((16,), jnp.int32) + col_a])
        if r_ref is not None:
            bb = plsc.load_gather(scal_ref, [l_idx, jnp.zeros((16,), jnp.int32) + col_b])
            r = r_ref[pl.ds(off, 16)]
            s = a + r * bb
        else:
            s = a
        if off == 34:
            s = jnp.where(lane >= 14, s, jnp.float32(_NEG))
        svecs.append(s)
    mv = jnp.maximum(jnp.maximum(svecs[0], svecs[1]),
                     jnp.maximum(svecs[2], svecs[3]))
    m = jnp.max(mv)
    evecs = [jnp.exp(s - m) for s in svecs]
    total = jnp.sum(evecs[0] + evecs[1] + evecs[2] + evecs[3])
    inv = jnp.ones((16,), jnp.float32) / (jnp.zeros((16,), jnp.float32) + total)
    return evecs, inv


def _weighted_sum(e_ref, evecs, inv):
    pairs = ([(0 + j, 0, j) for j in range(16)]
             + [(16 + j, 1, j) for j in range(16)]
             + [(32 + j, 2, j) for j in range(16)]
             + [(48, 3, 14), (49, 3, 15)])
    acc = [jnp.zeros((16,), jnp.float32) for _ in range(4)]
    for l, g, j in pairs:
        wl = evecs[g][j]
        for d in range(4):
            acc[d] = acc[d] + wl * e_ref[l, pl.ds(16 * d, 16)]
    return [a * inv for a in acc]


def _sc_body(uid_hbm, iid_hbm, hui_hbm, hur_hbm, soc_hbm, hii_hbm, hir_hbm,
             eu_hbm, ei_hbm, scal_hbm,
             h_hbm,
             uid_v, iid_v, uid_g, iid_g, ui_r, ur_r, fr_r, ii_r, ir_r,
             ue_v, ie_v, h_v,
             e1, s1, e2, s2, e3, s3,
             sem0, sem1, sem2, sem3, sem4, sem5, sem6):
    wid = lax.axis_index("s") * NC + lax.axis_index("c")
    base = wid * BPW
    sems = (sem0, sem1, sem2, sem3, sem4, sem5, sem6)

    pltpu.sync_copy(uid_hbm.at[pl.ds(base, BPW)], uid_g)
    pltpu.sync_copy(iid_hbm.at[pl.ds(base, BPW)], iid_g)
    pltpu.sync_copy(uid_hbm.at[pl.ds(base, BPW)], uid_v.at[pl.ds(0, BPW)])
    pltpu.sync_copy(iid_hbm.at[pl.ds(base, BPW)], iid_v.at[pl.ds(0, BPW)])
    g1 = pltpu.async_copy(eu_hbm.at[uid_g], ue_v, sems[5])
    g2 = pltpu.async_copy(ei_hbm.at[iid_g], ie_v, sems[6])
    g1.wait()
    g2.wait()

    def body(b, carry):
        uid_b = uid_v[pl.ds(b, 16)][0]
        iid_b = iid_v[pl.ds(b, 16)][0]
        hs = [
            pltpu.async_copy(hui_hbm.at[uid_b], ui_r, sems[0]),
            pltpu.async_copy(hur_hbm.at[uid_b], ur_r, sems[1]),
            pltpu.async_copy(soc_hbm.at[uid_b], fr_r, sems[2]),
            pltpu.async_copy(hii_hbm.at[iid_b], ii_r, sems[3]),
            pltpu.async_copy(hir_hbm.at[iid_b], ir_r, sems[4]),
        ]
        for h in hs:
            h.wait()
        gs = [
            pltpu.async_copy(ei_hbm.at[ui_r], e1, sems[0]),
            pltpu.async_copy(scal_hbm.at[ui_r], s1, sems[1]),
            pltpu.async_copy(eu_hbm.at[fr_r], e2, sems[2]),
            pltpu.async_copy(scal_hbm.at[fr_r], s2, sems[3]),
            pltpu.async_copy(ei_hbm.at[ii_r], e3, sems[4]),
            pltpu.async_copy(scal_hbm.at[ii_r], s3, sems[5]),
        ]
        for h in gs:
            h.wait()

        outs = []
        for e_ref, sc_ref, ca, cb, r_ref in (
                (e1, s1, 0, 1, ur_r),
                (e2, s2, 4, 4, None),
                (e3, s3, 2, 3, ir_r)):
            evecs, inv = _attention(sc_ref, ca, cb, r_ref)
            outs.append(_weighted_sum(e_ref, evecs, inv))

        o1, o2, o3 = outs
        for d in range(4):
            sl = pl.ds(16 * d, 16)
            h_v[b, sl] = ue_v[b, sl] + o1[d] + o2[d]
            h_v[b, pl.ds(D + 16 * d, 16)] = ie_v[b, sl] + o3[d]
        return carry

    lax.fori_loop(0, BPW, body, 0)
    pltpu.sync_copy(h_v, h_hbm.at[pl.ds(base, BPW)])


def _sc_attend(user_ids, item_ids, hui, hur, soc, hii, hir,
               embed_user, embed_item, scal):
    mesh = plsc.VectorSubcoreMesh(core_axis_name="c", subcore_axis_name="s",
                                  num_cores=NC, num_subcores=NS)
    f32, i32 = jnp.float32, jnp.int32
    kern = pl.kernel(
        _sc_body,
        out_type=jax.ShapeDtypeStruct((B, 2 * D), f32),
        mesh=mesh,
        scratch_types=[
            pltpu.VMEM((BPW + 16,), i32), pltpu.VMEM((BPW + 16,), i32),
            pltpu.VMEM((BPW,), i32), pltpu.VMEM((BPW,), i32),
            pltpu.VMEM((L,), i32), pltpu.VMEM((L,), f32),
            pltpu.VMEM((L,), i32), pltpu.VMEM((L,), i32),
            pltpu.VMEM((L,), f32),
            pltpu.VMEM((BPW, D), f32), pltpu.VMEM((BPW, D), f32),
            pltpu.VMEM((BPW, 2 * D), f32),
            pltpu.VMEM((L, D), f32), pltpu.VMEM((L, 8), f32),
            pltpu.VMEM((L, D), f32), pltpu.VMEM((L, 8), f32),
            pltpu.VMEM((L, D), f32), pltpu.VMEM((L, 8), f32),
        ] + [pltpu.SemaphoreType.DMA] * 7,
        compiler_params=pltpu.CompilerParams(
            needs_layout_passes=False, use_tc_tiling_on_sc=False),
    )
    return kern(user_ids, item_ids, hui, hur, soc, hii, hir,
                embed_user, embed_item, scal)


def _mlp_body(h_ref, w1, b1, w2, b2, o_ref):
    hh = jnp.maximum(h_ref[...] @ w1[...] + b1[...], 0.0)
    o_ref[...] = hh @ w2[...] + b2[...]


def _mlp(h, fc1_w, fc1_b, fc2_w, fc2_b):
    out = pl.pallas_call(
        _mlp_body,
        out_shape=jax.ShapeDtypeStruct((B, 1), jnp.float32),
    )(h, fc1_w, fc1_b.reshape(1, D), fc2_w, fc2_b.reshape(1, 1))
    return out.reshape(B)


def kernel(user_ids, item_ids, embed_user, embed_item, hist_u_items,
           hist_u_ratings, social_nbrs, hist_i_items, hist_i_ratings,
           W1, W2, W3, W4, W5, W6, W7, W8, W9, fc1_w, fc1_b, fc2_w, fc2_b):
    i32 = jnp.int32
    scal = _scal_tables(embed_item, embed_user, W2, W3, W5, W6, W8, W9)
    h = _sc_attend(user_ids.astype(i32), item_ids.astype(i32),
                   hist_u_items.astype(i32), hist_u_ratings,
                   social_nbrs.astype(i32), hist_i_items.astype(i32),
                   hist_i_ratings, embed_user, embed_item, scal)
    return _mlp(h, fc1_w, fc1_b, fc2_w, fc2_b)


# speedup vs baseline: 3.3534x; 1.0570x over previous
import functools

import jax
import jax.numpy as jnp
from jax import lax
from jax.experimental import pallas as pl
from jax.experimental.pallas import tpu as pltpu
from jax.experimental.pallas import tpu_sc as plsc

N_ROWS = 100000
D = 64
B = 4096
L = 50

NC, NS = 2, 16
NW = NC * NS
BPW = B // NW

_NEG = -1e30
_GROUPS = (0, 16, 32, 34)


def _scal_body(ei_ref, eu_ref, w2, w3, w5, w6, w8, w9, scal_ref):
    m23 = w2[...] @ w3[...]
    m89 = w8[...] @ w9[...]
    m56 = w5[...] @ w6[...]
    mi = jnp.concatenate([m23, w3[...], m89, w9[...]], axis=1)
    nb = ei_ref.shape[0]
    scal_ref[...] = jnp.concatenate(
        [ei_ref[...] @ mi, eu_ref[...] @ m56,
         jnp.zeros((nb, 3), jnp.float32)], axis=1)


def _scal_tables(embed_item, embed_user, W2, W3, W5, W6, W8, W9):
    nb = 5000
    grid = N_ROWS // nb
    full = lambda shape: pl.BlockSpec(shape, lambda i: (0, 0))
    return pl.pallas_call(
        _scal_body,
        grid=(grid,),
        in_specs=[
            pl.BlockSpec((nb, D), lambda i: (i, 0)),
            pl.BlockSpec((nb, D), lambda i: (i, 0)),
            full((D, D)), full((D, 1)), full((D, D)),
            full((D, 1)), full((D, D)), full((D, 1)),
        ],
        out_specs=pl.BlockSpec((nb, 8), lambda i: (i, 0)),
        out_shape=jax.ShapeDtypeStruct((N_ROWS, 8), jnp.float32),
    )(embed_item, embed_user, W2, W3, W5, W6, W8, W9)


def _attention(scal_ref, col_a, col_b, r_ref):
    lane = lax.iota(jnp.int32, 16)
    svecs = []
    for off in _GROUPS:
        l_idx = lane + off
        a = plsc.load_gather(scal_ref, [l_idx, jnp.zeros((16,), jnp.int32) + col_a])
        if r_ref is not None:
            bb = plsc.load_gather(scal_ref, [l_idx, jnp.zeros((16,), jnp.int32) + col_b])
            r = r_ref[pl.ds(off, 16)]
            s = a + r * bb
        else:
            s = a
        if off == 34:
            s = jnp.where(lane >= 14, s, jnp.float32(_NEG))
        svecs.append(s)
    mv = jnp.maximum(jnp.maximum(svecs[0], svecs[1]),
                     jnp.maximum(svecs[2], svecs[3]))
    m = jnp.max(mv)
    evecs = [jnp.exp(s - m) for s in svecs]
    total = jnp.sum(evecs[0] + evecs[1] + evecs[2] + evecs[3])
    inv = jnp.ones((16,), jnp.float32) / (jnp.zeros((16,), jnp.float32) + total)
    return evecs, inv


def _weighted_sum(e_ref, evecs, inv):
    pairs = ([(0 + j, 0, j) for j in range(16)]
             + [(16 + j, 1, j) for j in range(16)]
             + [(32 + j, 2, j) for j in range(16)]
             + [(48, 3, 14), (49, 3, 15)])
    acc = [jnp.zeros((16,), jnp.float32) for _ in range(4)]
    for l, g, j in pairs:
        wl = evecs[g][j]
        for d in range(4):
            acc[d] = acc[d] + wl * e_ref[l, pl.ds(16 * d, 16)]
    return [a * inv for a in acc]


def _sc_body(*refs):
    (uid_hbm, iid_hbm, hui_hbm, hur_hbm, soc_hbm, hii_hbm, hir_hbm,
     eu_hbm, ei_hbm, scal_hbm, h_hbm) = refs[:11]
    uid_v, iid_v, uid_g, iid_g, ue_v, ie_v, h_v = refs[11:18]
    hist = (refs[18:23], refs[23:28])
    gath = (refs[28:34], refs[34:40])
    hsem = (refs[40:45], refs[45:50])
    gsem = (refs[50:56], refs[56:62])
    psem = refs[62]

    wid = lax.axis_index("s") * NC + lax.axis_index("c")
    base = wid * BPW

    pltpu.sync_copy(uid_hbm.at[pl.ds(base, BPW)], uid_g)
    pltpu.sync_copy(iid_hbm.at[pl.ds(base, BPW)], iid_g)
    pltpu.sync_copy(uid_hbm.at[pl.ds(base, BPW)], uid_v.at[pl.ds(0, BPW)])
    pltpu.sync_copy(iid_hbm.at[pl.ds(base, BPW)], iid_v.at[pl.ds(0, BPW)])
    g1 = pltpu.async_copy(eu_hbm.at[uid_g], ue_v, psem)
    g2 = pltpu.async_copy(ei_hbm.at[iid_g], ie_v, psem)
    g1.wait()
    g2.wait()

    hist_tabs = (hui_hbm, hur_hbm, soc_hbm, hii_hbm, hir_hbm)

    def hist_issue(k, s):
        uk = uid_v[pl.ds(k, 16)][0]
        ik = iid_v[pl.ds(k, 16)][0]
        for t, (tab, idx) in enumerate(
                ((hui_hbm, uk), (hur_hbm, uk), (soc_hbm, uk),
                 (hii_hbm, ik), (hir_hbm, ik))):
            pltpu.async_copy(tab.at[idx], hist[s][t], hsem[s][t])

    def hist_wait(s):
        for t, tab in enumerate(hist_tabs):
            pltpu.make_async_copy(tab.at[0], hist[s][t], hsem[s][t]).wait()

    def gath_specs(s):
        ui_r, _, fr_r, ii_r, _ = hist[s]
        return ((ei_hbm, ui_r, gath[s][0]), (scal_hbm, ui_r, gath[s][1]),
                (eu_hbm, fr_r, gath[s][2]), (scal_hbm, fr_r, gath[s][3]),
                (ei_hbm, ii_r, gath[s][4]), (scal_hbm, ii_r, gath[s][5]))

    def gath_issue(s):
        for t, (tab, idx, dst) in enumerate(gath_specs(s)):
            pltpu.async_copy(tab.at[idx], dst, gsem[s][t])

    def gath_wait(s):
        for t, (tab, idx, dst) in enumerate(gath_specs(s)):
            pltpu.make_async_copy(tab.at[idx], dst, gsem[s][t]).wait()

    def compute(bb, s):
        e1, s1, e2, s2, e3, s3 = gath[s]
        _, ur_r, _, _, ir_r = hist[s]
        outs = []
        for e_ref, sc_ref, ca, cb, r_ref in (
                (e1, s1, 0, 1, ur_r),
                (e2, s2, 4, 4, None),
                (e3, s3, 2, 3, ir_r)):
            evecs, inv = _attention(sc_ref, ca, cb, r_ref)
            outs.append(_weighted_sum(e_ref, evecs, inv))
        o1, o2, o3 = outs
        for d in range(4):
            sl = pl.ds(16 * d, 16)
            h_v[bb, sl] = ue_v[bb, sl] + o1[d] + o2[d]
            h_v[bb, pl.ds(D + 16 * d, 16)] = ie_v[bb, sl] + o3[d]

    hist_issue(0, 0)
    hist_wait(0)
    gath_issue(0)
    hist_issue(1, 1)

    def body(i, carry):
        for s in (0, 1):
            bb = 2 * i + s
            ns = 1 - s

            @pl.when(bb + 1 < BPW)
            def _():
                hist_wait(ns)
                gath_issue(ns)

            gath_wait(s)

            @pl.when(bb + 2 < BPW)
            def _():
                hist_issue(bb + 2, s)

            compute(bb, s)
        return carry

    lax.fori_loop(0, BPW // 2, body, 0)
    pltpu.sync_copy(h_v, h_hbm.at[pl.ds(base, BPW)])


def _sc_attend(user_ids, item_ids, hui, hur, soc, hii, hir,
               embed_user, embed_item, scal):
    mesh = plsc.VectorSubcoreMesh(core_axis_name="c", subcore_axis_name="s",
                                  num_cores=NC, num_subcores=NS)
    f32, i32 = jnp.float32, jnp.int32
    hist_slot = [pltpu.VMEM((L,), i32), pltpu.VMEM((L,), f32),
                 pltpu.VMEM((L,), i32), pltpu.VMEM((L,), i32),
                 pltpu.VMEM((L,), f32)]
    gath_slot = [pltpu.VMEM((L, D), f32), pltpu.VMEM((L, 8), f32),
                 pltpu.VMEM((L, D), f32), pltpu.VMEM((L, 8), f32),
                 pltpu.VMEM((L, D), f32), pltpu.VMEM((L, 8), f32)]
    kern = pl.kernel(
        _sc_body,
        out_type=jax.ShapeDtypeStruct((B, 2 * D), f32),
        mesh=mesh,
        scratch_types=(
            [pltpu.VMEM((BPW + 16,), i32), pltpu.VMEM((BPW + 16,), i32),
             pltpu.VMEM((BPW,), i32), pltpu.VMEM((BPW,), i32),
             pltpu.VMEM((BPW, D), f32), pltpu.VMEM((BPW, D), f32),
             pltpu.VMEM((BPW, 2 * D), f32)]
            + hist_slot + hist_slot + gath_slot + gath_slot
            + [pltpu.SemaphoreType.DMA] * 23
        ),
        compiler_params=pltpu.CompilerParams(
            needs_layout_passes=False, use_tc_tiling_on_sc=False),
    )
    return kern(user_ids, item_ids, hui, hur, soc, hii, hir,
                embed_user, embed_item, scal)


def _mlp_body(h_ref, w1, b1, w2, b2, o_ref):
    hh = jnp.maximum(h_ref[...] @ w1[...] + b1[...], 0.0)
    o_ref[...] = hh @ w2[...] + b2[...]


def _mlp(h, fc1_w, fc1_b, fc2_w, fc2_b):
    out = pl.pallas_call(
        _mlp_body,
        out_shape=jax.ShapeDtypeStruct((B, 1), jnp.float32),
    )(h, fc1_w, fc1_b.reshape(1, D), fc2_w, fc2_b.reshape(1, 1))
    return out.reshape(B)


def kernel(user_ids, item_ids, embed_user, embed_item, hist_u_items,
           hist_u_ratings, social_nbrs, hist_i_items, hist_i_ratings,
           W1, W2, W3, W4, W5, W6, W7, W8, W9, fc1_w, fc1_b, fc2_w, fc2_b):
    i32 = jnp.int32
    scal = _scal_tables(embed_item, embed_user, W2, W3, W5, W6, W8, W9)
    h = _sc_attend(user_ids.astype(i32), item_ids.astype(i32),
                   hist_u_items.astype(i32), hist_u_ratings,
                   social_nbrs.astype(i32), hist_i_items.astype(i32),
                   hist_i_ratings, embed_user, embed_item, scal)
    return _mlp(h, fc1_w, fc1_b, fc2_w, fc2_b)


# speedup vs baseline: 3.9712x; 1.1842x over previous
import functools

import jax
import jax.numpy as jnp
from jax import lax
from jax.experimental import pallas as pl
from jax.experimental.pallas import tpu as pltpu
from jax.experimental.pallas import tpu_sc as plsc

N_ROWS = 100000
D = 64
B = 4096
L = 50

NC, NS = 2, 16
NW = NC * NS
BPW = B // NW

_NEG = -1e30
_GROUPS = (0, 16, 32, 34)


def _scal_body(ei_ref, eu_ref, w2, w3, w5, w6, w8, w9, scal_ref):
    m23 = w2[...] @ w3[...]
    m89 = w8[...] @ w9[...]
    m56 = w5[...] @ w6[...]
    mi = jnp.concatenate([m23, w3[...], m89, w9[...]], axis=1)
    nb = ei_ref.shape[0]
    scal_ref[...] = jnp.concatenate(
        [ei_ref[...] @ mi, eu_ref[...] @ m56,
         jnp.zeros((nb, 3), jnp.float32)], axis=1)


def _scal_tables(embed_item, embed_user, W2, W3, W5, W6, W8, W9):
    nb = 5000
    grid = N_ROWS // nb
    full = lambda shape: pl.BlockSpec(shape, lambda i: (0, 0))
    return pl.pallas_call(
        _scal_body,
        grid=(grid,),
        in_specs=[
            pl.BlockSpec((nb, D), lambda i: (i, 0)),
            pl.BlockSpec((nb, D), lambda i: (i, 0)),
            full((D, D)), full((D, 1)), full((D, D)),
            full((D, 1)), full((D, D)), full((D, 1)),
        ],
        out_specs=pl.BlockSpec((nb, 8), lambda i: (i, 0)),
        out_shape=jax.ShapeDtypeStruct((N_ROWS, 8), jnp.float32),
    )(embed_item, embed_user, W2, W3, W5, W6, W8, W9)


def _attention(scal_ref, col_a, col_b, r_ref, bb, d0):
    lane = lax.iota(jnp.int32, 16)
    svecs = []
    for off in _GROUPS:
        l_idx = lane + off
        a = plsc.load_gather(scal_ref, [l_idx, jnp.zeros((16,), jnp.int32) + col_a])
        if r_ref is not None:
            bb_ = plsc.load_gather(scal_ref, [l_idx, jnp.zeros((16,), jnp.int32) + col_b])
            r = r_ref[bb, pl.ds(d0 + off, 16)]
            s = a + r * bb_
        else:
            s = a
        if off == 34:
            s = jnp.where(lane >= 14, s, jnp.float32(_NEG))
        svecs.append(s)
    mv = jnp.maximum(jnp.maximum(svecs[0], svecs[1]),
                     jnp.maximum(svecs[2], svecs[3]))
    m = jnp.max(mv)
    evecs = [jnp.exp(s - m) for s in svecs]
    total = jnp.sum(evecs[0] + evecs[1] + evecs[2] + evecs[3])
    inv = jnp.ones((16,), jnp.float32) / (jnp.zeros((16,), jnp.float32) + total)
    return evecs, inv


def _weighted_sum(e_ref, evecs, inv):
    pairs = ([(0 + j, 0, j) for j in range(16)]
             + [(16 + j, 1, j) for j in range(16)]
             + [(32 + j, 2, j) for j in range(16)]
             + [(48, 3, 14), (49, 3, 15)])
    acc = [jnp.zeros((16,), jnp.float32) for _ in range(4)]
    for l, g, j in pairs:
        wl = evecs[g][j]
        for d in range(4):
            acc[d] = acc[d] + wl * e_ref[l, pl.ds(16 * d, 16)]
    return [a * inv for a in acc]


def _sc_body(*refs):
    (uid_hbm, iid_hbm, hui_hbm, hur_hbm, soc_hbm, hii_hbm, hir_hbm,
     eu_hbm, ei_hbm, scal_hbm, h_hbm) = refs[:11]
    uid_v, iid_v, uid_g, iid_g, ue_v, ie_v, h_v = refs[11:18]
    ui_w, ur_w, fr_w, ii_w, ir_w = refs[18:23]
    idxs = (refs[23:26], refs[26:29])
    gath = (refs[29:35], refs[35:41])
    hsem = refs[41:46]
    gsem = (refs[46:52], refs[52:58])
    psem = refs[58]

    wid = lax.axis_index("s") * NC + lax.axis_index("c")
    base = wid * BPW

    pltpu.sync_copy(uid_hbm.at[pl.ds(base, BPW)], uid_g)
    pltpu.sync_copy(iid_hbm.at[pl.ds(base, BPW)], iid_g)
    pltpu.sync_copy(uid_hbm.at[pl.ds(base, BPW)], uid_v.at[pl.ds(0, BPW)])
    pltpu.sync_copy(iid_hbm.at[pl.ds(base, BPW)], iid_v.at[pl.ds(0, BPW)])
    g1 = pltpu.async_copy(eu_hbm.at[uid_g], ue_v, psem)
    g2 = pltpu.async_copy(ei_hbm.at[iid_g], ie_v, psem)

    def ids_at(k):
        return uid_v[pl.ds(k, 16)][0], iid_v[pl.ds(k, 16)][0]

    hist_specs = ((hui_hbm, ui_w, 0), (hur_hbm, ur_w, 0), (soc_hbm, fr_w, 0),
                  (hii_hbm, ii_w, 1), (hir_hbm, ir_w, 1))

    def wbase(start):
        return pl.multiple_of(jnp.minimum(start & -8, N_ROWS * L - 64), 8)

    def hfetch(k, carry):
        uk, ik = ids_at(k)
        b_u = wbase(uk * L)
        b_i = wbase(ik * L)
        for t, (tab, dst, which) in enumerate(hist_specs):
            pltpu.async_copy(tab.at[pl.ds(b_i if which else b_u, 64)],
                             dst.at[k], hsem[t])
        return carry

    lax.fori_loop(0, BPW, hfetch, 0)

    def hdrain(k, carry):
        for t, (tab, dst, which) in enumerate(hist_specs):
            pltpu.make_async_copy(tab.at[pl.ds(0, 64)], dst.at[k],
                                  hsem[t]).wait()
        return carry

    lax.fori_loop(0, BPW, hdrain, 0)
    g1.wait()
    g2.wait()

    def gath_issue(k, s):
        uk, ik = ids_at(k)
        d_u = uk * L - wbase(uk * L)
        d_i = ik * L - wbase(ik * L)
        for off in _GROUPS:
            sl = pl.ds(off, 16)
            idxs[s][0][sl] = ui_w[k, pl.ds(d_u + off, 16)]
            idxs[s][1][sl] = fr_w[k, pl.ds(d_u + off, 16)]
            idxs[s][2][sl] = ii_w[k, pl.ds(d_i + off, 16)]
        for t, (tab, idx, dst) in enumerate(_gspecs(s, idxs, gath,
                                                    ei_hbm, eu_hbm, scal_hbm)):
            pltpu.async_copy(tab.at[idx], dst, gsem[s][t])

    def gath_wait(s):
        for t, (tab, idx, dst) in enumerate(_gspecs(s, idxs, gath,
                                                    ei_hbm, eu_hbm, scal_hbm)):
            pltpu.make_async_copy(tab.at[idx], dst, gsem[s][t]).wait()

    def compute(bb, s):
        e1, s1, e2, s2, e3, s3 = gath[s]
        uk, ik = ids_at(bb)
        d_u = uk * L - wbase(uk * L)
        d_i = ik * L - wbase(ik * L)
        outs = []
        for e_ref, sc_ref, ca, cb, r_ref, d0 in (
                (e1, s1, 0, 1, ur_w, d_u),
                (e2, s2, 4, 4, None, 0),
                (e3, s3, 2, 3, ir_w, d_i)):
            evecs, inv = _attention(sc_ref, ca, cb, r_ref, bb, d0)
            outs.append(_weighted_sum(e_ref, evecs, inv))
        o1, o2, o3 = outs
        for d in range(4):
            sl = pl.ds(16 * d, 16)
            h_v[bb, sl] = ue_v[bb, sl] + o1[d] + o2[d]
            h_v[bb, pl.ds(D + 16 * d, 16)] = ie_v[bb, sl] + o3[d]

    gath_issue(0, 0)

    def body(i, carry):
        for s in (0, 1):
            bb = 2 * i + s
            ns = 1 - s

            @pl.when(bb + 1 < BPW)
            def _():
                gath_issue(bb + 1, ns)

            gath_wait(s)
            compute(bb, s)
        return carry

    lax.fori_loop(0, BPW // 2, body, 0)
    pltpu.sync_copy(h_v, h_hbm.at[pl.ds(base, BPW)])


def _gspecs(s, idxs, gath, ei_hbm, eu_hbm, scal_hbm):
    i1, i2, i3 = idxs[s]
    return ((ei_hbm, i1, gath[s][0]), (scal_hbm, i1, gath[s][1]),
            (eu_hbm, i2, gath[s][2]), (scal_hbm, i2, gath[s][3]),
            (ei_hbm, i3, gath[s][4]), (scal_hbm, i3, gath[s][5]))


def _sc_attend(user_ids, item_ids, hui, hur, soc, hii, hir,
               embed_user, embed_item, scal):
    mesh = plsc.VectorSubcoreMesh(core_axis_name="c", subcore_axis_name="s",
                                  num_cores=NC, num_subcores=NS)
    f32, i32 = jnp.float32, jnp.int32
    idx_slot = [pltpu.VMEM((L,), i32)] * 3
    gath_slot = [pltpu.VMEM((L, D), f32), pltpu.VMEM((L, 8), f32),
                 pltpu.VMEM((L, D), f32), pltpu.VMEM((L, 8), f32),
                 pltpu.VMEM((L, D), f32), pltpu.VMEM((L, 8), f32)]
    kern = pl.kernel(
        _sc_body,
        out_type=jax.ShapeDtypeStruct((B, 2 * D), f32),
        mesh=mesh,
        scratch_types=(
            [pltpu.VMEM((BPW + 16,), i32), pltpu.VMEM((BPW + 16,), i32),
             pltpu.VMEM((BPW,), i32), pltpu.VMEM((BPW,), i32),
             pltpu.VMEM((BPW, D), f32), pltpu.VMEM((BPW, D), f32),
             pltpu.VMEM((BPW, 2 * D), f32),
             pltpu.VMEM((BPW, 64), i32), pltpu.VMEM((BPW, 64), f32),
             pltpu.VMEM((BPW, 64), i32), pltpu.VMEM((BPW, 64), i32),
             pltpu.VMEM((BPW, 64), f32)]
            + idx_slot + idx_slot + gath_slot + gath_slot
            + [pltpu.SemaphoreType.DMA] * 18
        ),
        compiler_params=pltpu.CompilerParams(
            needs_layout_passes=False, use_tc_tiling_on_sc=False),
    )
    return kern(user_ids, item_ids, hui, hur, soc, hii, hir,
                embed_user, embed_item, scal)


def _mlp_body(h_ref, w1, b1, w2, b2, o_ref):
    hh = jnp.maximum(h_ref[...] @ w1[...] + b1[...], 0.0)
    o_ref[...] = hh @ w2[...] + b2[...]


def _mlp(h, fc1_w, fc1_b, fc2_w, fc2_b):
    out = pl.pallas_call(
        _mlp_body,
        out_shape=jax.ShapeDtypeStruct((B, 1), jnp.float32),
    )(h, fc1_w, fc1_b.reshape(1, D), fc2_w, fc2_b.reshape(1, 1))
    return out.reshape(B)


def kernel(user_ids, item_ids, embed_user, embed_item, hist_u_items,
           hist_u_ratings, social_nbrs, hist_i_items, hist_i_ratings,
           W1, W2, W3, W4, W5, W6, W7, W8, W9, fc1_w, fc1_b, fc2_w, fc2_b):
    i32 = jnp.int32
    scal = _scal_tables(embed_item, embed_user, W2, W3, W5, W6, W8, W9)
    h = _sc_attend(user_ids.astype(i32), item_ids.astype(i32),
                   hist_u_items.astype(i32).reshape(-1),
                   hist_u_ratings.reshape(-1),
                   social_nbrs.astype(i32).reshape(-1),
                   hist_i_items.astype(i32).reshape(-1),
                   hist_i_ratings.reshape(-1),
                   embed_user, embed_item, scal)
    return _mlp(h, fc1_w, fc1_b, fc2_w, fc2_b)
